# Initial kernel scaffold; baseline (speedup 1.0000x reference)
import jax
import jax.numpy as jnp
from jax.experimental import pallas as pl


def kernel(x, edge_index, W1, a1_src, a1_dst, b1, W2, a2_src, a2_dst, b2, Wm, am_src, am_dst, bm, Wi, ai_src, ai_dst, bi, Wc, ac_src, ac_dst, bc):
    raise NotImplementedError("write your pallas kernel here")


# speedup vs baseline: 1.1686x; 1.1686x over previous
import functools

import jax
import jax.numpy as jnp
from jax.experimental import pallas as pl


def _dense_body(x_ref, w_ref, asrc_ref, adst_ref, h_ref, as_ref, ad_ref):
    h = jnp.dot(x_ref[...], w_ref[...], preferred_element_type=jnp.float32)
    h_ref[...] = h
    B = h.shape[0]
    heads, hu = asrc_ref.shape
    h3 = h.reshape(B, heads, hu)
    as_ref[...] = jnp.sum(h3 * asrc_ref[...][None], axis=-1)
    ad_ref[...] = jnp.sum(h3 * adst_ref[...][None], axis=-1)


def _dense(x, W, a_src, a_dst):
    N, D = x.shape
    F = W.shape[1]
    heads, hu = a_src.shape
    B = 1000
    grid = (N // B,)
    h, als, ald = pl.pallas_call(
        _dense_body,
        grid=grid,
        in_specs=[
            pl.BlockSpec((B, D), lambda i: (i, 0)),
            pl.BlockSpec((D, F), lambda i: (0, 0)),
            pl.BlockSpec((heads, hu), lambda i: (0, 0)),
            pl.BlockSpec((heads, hu), lambda i: (0, 0)),
        ],
        out_specs=[
            pl.BlockSpec((B, F), lambda i: (i, 0)),
            pl.BlockSpec((B, heads), lambda i: (i, 0)),
            pl.BlockSpec((B, heads), lambda i: (i, 0)),
        ],
        out_shape=[
            jax.ShapeDtypeStruct((N, F), jnp.float32),
            jax.ShapeDtypeStruct((N, heads), jnp.float32),
            jax.ShapeDtypeStruct((N, heads), jnp.float32),
        ],
    )(x, W, a_src, a_dst)
    return h, als, ald


def _gat_conv(x, src, dst, W, a_src, a_dst, b):
    Nn = x.shape[0]
    heads, hu = a_src.shape
    h, als, ald = _dense(x, W, a_src, a_dst)
    shift = jax.nn.leaky_relu(
        jnp.max(als, axis=0) + jnp.max(ald, axis=0), negative_slope=0.2)
    e = jax.nn.leaky_relu(als[src] + ald[dst], negative_slope=0.2)
    ex = jnp.exp(e - shift[None, :])
    denom = jax.ops.segment_sum(ex, dst, num_segments=Nn)
    h3 = h.reshape(Nn, heads, hu)
    msg = h3[src] * ex[:, :, None]
    num = jax.ops.segment_sum(msg, dst, num_segments=Nn)
    out = num / (denom[:, :, None] + 1e-16)
    return out.reshape(Nn, heads * hu) + b


def kernel(x, edge_index, W1, a1_src, a1_dst, b1, W2, a2_src, a2_dst, b2,
           Wm, am_src, am_dst, bm, Wi, ai_src, ai_dst, bi, Wc, ac_src, ac_dst, bc):
    src = edge_index[0]
    dst = edge_index[1]
    x1 = jax.nn.elu(_gat_conv(x, src, dst, W1, a1_src, a1_dst, b1))
    x2 = jax.nn.elu(_gat_conv(x1, src, dst, W2, a2_src, a2_dst, b2))
    med = _gat_conv(x2, src, dst, Wm, am_src, am_dst, bm)
    imm = _gat_conv(x2, src, dst, Wi, ai_src, ai_dst, bi)
    care = _gat_conv(x2, src, dst, Wc, ac_src, ac_dst, bc)
    return (med, imm, care)


# speedup vs baseline: 1.1722x; 1.0031x over previous
import functools

import jax
import jax.numpy as jnp
from jax.experimental import pallas as pl


def _dense_body(x_ref, w_ref, asrc_ref, adst_ref, h_ref, as_ref, ad_ref,
                ms_ref, md_ref):
    i = pl.program_id(0)
    h = jnp.dot(x_ref[...], w_ref[...], preferred_element_type=jnp.float32)
    h_ref[...] = h
    B = h.shape[0]
    heads, hu = asrc_ref.shape
    h3 = h.reshape(B, heads, hu)
    als = jnp.sum(h3 * asrc_ref[...][None], axis=-1)
    ald = jnp.sum(h3 * adst_ref[...][None], axis=-1)
    as_ref[...] = als
    ad_ref[...] = ald
    ms = jnp.max(als, axis=0, keepdims=True)
    md = jnp.max(ald, axis=0, keepdims=True)

    @pl.when(i == 0)
    def _():
        ms_ref[...] = ms
        md_ref[...] = md

    @pl.when(i != 0)
    def _():
        ms_ref[...] = jnp.maximum(ms_ref[...], ms)
        md_ref[...] = jnp.maximum(md_ref[...], md)


def _dense(x, W, a_src, a_dst):
    N, D = x.shape
    F = W.shape[1]
    heads, hu = a_src.shape
    B = 1000
    grid = (N // B,)
    return pl.pallas_call(
        _dense_body,
        grid=grid,
        in_specs=[
            pl.BlockSpec((B, D), lambda i: (i, 0)),
            pl.BlockSpec((D, F), lambda i: (0, 0)),
            pl.BlockSpec((heads, hu), lambda i: (0, 0)),
            pl.BlockSpec((heads, hu), lambda i: (0, 0)),
        ],
        out_specs=[
            pl.BlockSpec((B, F), lambda i: (i, 0)),
            pl.BlockSpec((B, heads), lambda i: (i, 0)),
            pl.BlockSpec((B, heads), lambda i: (i, 0)),
            pl.BlockSpec((1, heads), lambda i: (0, 0)),
            pl.BlockSpec((1, heads), lambda i: (0, 0)),
        ],
        out_shape=[
            jax.ShapeDtypeStruct((N, F), jnp.float32),
            jax.ShapeDtypeStruct((N, heads), jnp.float32),
            jax.ShapeDtypeStruct((N, heads), jnp.float32),
            jax.ShapeDtypeStruct((1, heads), jnp.float32),
            jax.ShapeDtypeStruct((1, heads), jnp.float32),
        ],
    )(x, W, a_src, a_dst)


def _norm_body(num_ref, den_ref, b_ref, y_ref, *, apply_elu, hu):
    num = num_ref[...]
    B, F = num.shape
    heads = F // hu
    den = jnp.broadcast_to(den_ref[...][:, :, None], (B, heads, hu))
    y = num / (den.reshape(B, F) + 1e-16) + b_ref[...]
    if apply_elu:
        y = jnp.where(y > 0, y, jnp.exp(jnp.minimum(y, 0.0)) - 1.0)
    y_ref[...] = y


def _norm(num, den, b, apply_elu, hu):
    N, F = num.shape
    heads = F // hu
    B = 1000
    grid = (N // B,)
    return pl.pallas_call(
        functools.partial(_norm_body, apply_elu=apply_elu, hu=hu),
        grid=grid,
        in_specs=[
            pl.BlockSpec((B, F), lambda i: (i, 0)),
            pl.BlockSpec((B, heads), lambda i: (i, 0)),
            pl.BlockSpec((1, F), lambda i: (0, 0)),
        ],
        out_specs=pl.BlockSpec((B, F), lambda i: (i, 0)),
        out_shape=jax.ShapeDtypeStruct((N, F), jnp.float32),
    )(num, den, b.reshape(1, F))


def _gat_conv(x, src, dst, W, a_src, a_dst, b, apply_elu):
    Nn = x.shape[0]
    heads, hu = a_src.shape
    h, als, ald, ms, md = _dense(x, W, a_src, a_dst)
    z = ms[0] + md[0]
    shift = jnp.where(z >= 0, z, 0.2 * z)
    e = als[src] + ald[dst]
    e = jnp.where(e >= 0, e, 0.2 * e)
    ex = jnp.exp(e - shift[None, :])
    den = jax.ops.segment_sum(ex, dst, num_segments=Nn)
    h3 = h.reshape(Nn, heads, hu)
    msg = h3[src] * ex[:, :, None]
    num = jax.ops.segment_sum(msg, dst, num_segments=Nn).reshape(Nn, heads * hu)
    return _norm(num, den, b, apply_elu, hu)


def kernel(x, edge_index, W1, a1_src, a1_dst, b1, W2, a2_src, a2_dst, b2,
           Wm, am_src, am_dst, bm, Wi, ai_src, ai_dst, bi, Wc, ac_src, ac_dst, bc):
    src = edge_index[0]
    dst = edge_index[1]
    x1 = _gat_conv(x, src, dst, W1, a1_src, a1_dst, b1, True)
    x2 = _gat_conv(x1, src, dst, W2, a2_src, a2_dst, b2, True)
    med = _gat_conv(x2, src, dst, Wm, am_src, am_dst, bm, False)
    imm = _gat_conv(x2, src, dst, Wi, ai_src, ai_dst, bi, False)
    care = _gat_conv(x2, src, dst, Wc, ac_src, ac_dst, bc, False)
    return (med, imm, care)
